# Initial kernel scaffold; baseline (speedup 1.0000x reference)
import jax
import jax.numpy as jnp
from jax.experimental import pallas as pl


def kernel(x, edge_index, W1, b1, W2, b2):
    raise NotImplementedError("write your pallas kernel here")


# speedup vs baseline: 2.4492x; 2.4492x over previous
import functools

import jax
import jax.numpy as jnp
from jax import lax
from jax.experimental import pallas as pl
from jax.experimental.pallas import tpu as pltpu
from jax.experimental.pallas import tpu_sc as plsc

F32 = jnp.float32
NTILES = 32
NSUB = 16
BATCH = 128
CHUNK = 128
R = 512


def _cdiv(a, b):
    return (a + b - 1) // b


def _dinv_from(degp_ref):
    dp = degp_ref[0] + degp_ref[1]
    return lax.rsqrt(1.0 + dp[:, 0:1])


def _mm1_body(degp_ref, x_ref, w_ref, *out_refs):
    dinv = _dinv_from(degp_ref)
    h = jnp.dot(x_ref[...], w_ref[...], preferred_element_type=F32)
    m = h * dinv
    for c, o in enumerate(out_refs):
        o[...] = m[:, c * CHUNK:(c + 1) * CHUNK]


def _mm2_body(nch_in, nch_out, degp_ref, b_ref, w_ref, *refs):
    m_refs = refs[:nch_in]
    a_refs = refs[nch_in:2 * nch_in]
    out_refs = refs[2 * nch_in:]
    dinv = _dinv_from(degp_ref)
    parts = [m_refs[c][...] + a_refs[c][0] + a_refs[c][1] for c in range(nch_in)]
    t = jnp.concatenate(parts, axis=1)
    y = jnp.maximum(t * dinv + b_ref[...], 0.0)
    m2 = jnp.dot(y, w_ref[...], preferred_element_type=F32) * dinv
    for c, o in enumerate(out_refs):
        o[...] = m2[:, c * CHUNK:(c + 1) * CHUNK]


def _mm3_body(nch, degp_ref, b_ref, *refs):
    m_refs = refs[:nch]
    a_refs = refs[nch:2 * nch]
    out_ref = refs[2 * nch]
    dinv = _dinv_from(degp_ref)
    parts = [m_refs[c][...] + a_refs[c][0] + a_refs[c][1] for c in range(nch)]
    t = jnp.concatenate(parts, axis=1)
    out_ref[...] = t * dinv + b_ref[...]


def _sc_mesh():
    return plsc.VectorSubcoreMesh(core_axis_name="c", subcore_axis_name="s")


def _make_deg(NP, nbt):
    KS = NP // NSUB // BATCH

    def body(dst_r, ones_hbm, zeros_hbm, out_ref, idx_d, ones_v,
             bufA, acc):
        cid = lax.axis_index("c")
        sid = lax.axis_index("s")
        wid = cid * NSUB + sid
        pltpu.sync_copy(dst_r.at[wid], idx_d)
        pltpu.sync_copy(ones_hbm, ones_v)
        pltpu.sync_copy(zeros_hbm, bufA)
        base = sid * (NP // NSUB)
        for k in range(KS):
            pltpu.sync_copy(bufA, acc.at[pl.ds(base + k * BATCH, BATCH)])
        plsc.subcore_barrier()

        def step(j, carry):
            pltpu.sync_copy(ones_v, acc.at[idx_d.at[j]], add=True)
            return carry

        lax.fori_loop(0, nbt, step, 0)
        plsc.subcore_barrier()
        for k in range(KS):
            pltpu.sync_copy(acc.at[pl.ds(base + k * BATCH, BATCH)], bufA)
            pltpu.sync_copy(bufA, out_ref.at[cid, pl.ds(base + k * BATCH, BATCH)])

    return pl.kernel(
        body,
        out_type=jax.ShapeDtypeStruct((2, NP, CHUNK), F32),
        mesh=_sc_mesh(),
        scratch_types=[
            pltpu.VMEM((nbt, BATCH), jnp.int32),
            pltpu.VMEM((BATCH, CHUNK), F32),
            pltpu.VMEM((BATCH, CHUNK), F32),
            pltpu.VMEM_SHARED((NP, CHUNK), F32),
        ],
    )


def _make_agg(NP, nbt, nch):
    KS = NP // NSUB // BATCH

    def body(src_r, dst_r, zeros_hbm, *rest):
        m_refs = rest[:nch]
        out_refs = rest[nch:2 * nch]
        idx_s, idx_d, zeros_v, bufA, acc = rest[2 * nch:]
        cid = lax.axis_index("c")
        sid = lax.axis_index("s")
        wid = cid * NSUB + sid
        pltpu.sync_copy(src_r.at[wid], idx_s)
        pltpu.sync_copy(dst_r.at[wid], idx_d)
        pltpu.sync_copy(zeros_hbm, zeros_v)
        base = sid * (NP // NSUB)
        for ch in range(nch):
            for k in range(KS):
                pltpu.sync_copy(zeros_v, acc.at[pl.ds(base + k * BATCH, BATCH)])
            plsc.subcore_barrier()

            def step(j, carry, _m=m_refs[ch]):
                pltpu.sync_copy(_m.at[idx_s.at[j]], bufA)
                pltpu.sync_copy(bufA, acc.at[idx_d.at[j]], add=True)
                return carry

            lax.fori_loop(0, nbt, step, 0)
            plsc.subcore_barrier()
            for k in range(KS):
                pltpu.sync_copy(acc.at[pl.ds(base + k * BATCH, BATCH)], bufA)
                pltpu.sync_copy(
                    bufA, out_refs[ch].at[cid, pl.ds(base + k * BATCH, BATCH)])
            plsc.subcore_barrier()

    return pl.kernel(
        body,
        out_type=[jax.ShapeDtypeStruct((2, NP, CHUNK), F32)] * nch,
        mesh=_sc_mesh(),
        scratch_types=[
            pltpu.VMEM((nbt, BATCH), jnp.int32),
            pltpu.VMEM((nbt, BATCH), jnp.int32),
            pltpu.VMEM((BATCH, CHUNK), F32),
            pltpu.VMEM((BATCH, CHUNK), F32),
            pltpu.VMEM_SHARED((NP, CHUNK), F32),
        ],
    )


def kernel(x, edge_index, W1, b1, W2, b2):
    N, DI = x.shape
    DH = W1.shape[1]
    DO = W2.shape[1]
    E = edge_index.shape[1]
    nch1 = DH // CHUNK
    nch2 = DO // CHUNK

    NP = _cdiv(N + 1, NSUB * BATCH) * (NSUB * BATCH)
    nbt = _cdiv(E, NTILES * BATCH)
    EP = NTILES * BATCH * nbt
    grid = (NP // R,)

    src = jnp.concatenate([edge_index[0],
                           jnp.full((EP - E,), N, jnp.int32)]).reshape(
                               NTILES, nbt, BATCH)
    dst = jnp.concatenate([edge_index[1],
                           jnp.full((EP - E,), N, jnp.int32)]).reshape(
                               NTILES, nbt, BATCH)
    xp = jnp.pad(x, ((0, NP - N), (0, 0)))
    zerosC = jnp.zeros((BATCH, CHUNK), F32)
    onesC = jnp.ones((BATCH, CHUNK), F32)

    degp = _make_deg(NP, nbt)(dst, onesC, zerosC)


    degp_spec = pl.BlockSpec((2, R, CHUNK), lambda i: (0, i, 0))
    chunk_spec = pl.BlockSpec((R, CHUNK), lambda i: (i, 0))
    agg_spec = pl.BlockSpec((2, R, CHUNK), lambda i: (0, i, 0))
    m1 = pl.pallas_call(
        _mm1_body,
        grid=grid,
        in_specs=[degp_spec,
                  pl.BlockSpec((R, DI), lambda i: (i, 0)),
                  pl.BlockSpec((DI, DH), lambda i: (0, 0))],
        out_specs=[chunk_spec] * nch1,
        out_shape=[jax.ShapeDtypeStruct((NP, CHUNK), F32)] * nch1,
    )(degp, xp, W1)

    agg1 = _make_agg(NP, nbt, nch1)(src, dst, zerosC, *m1)

    m2 = pl.pallas_call(
        functools.partial(_mm2_body, nch1, nch2),
        grid=grid,
        in_specs=[degp_spec,
                  pl.BlockSpec((1, DH), lambda i: (0, 0)),
                  pl.BlockSpec((DH, DO), lambda i: (0, 0))]
                 + [chunk_spec] * nch1 + [agg_spec] * nch1,
        out_specs=[chunk_spec] * nch2,
        out_shape=[jax.ShapeDtypeStruct((NP, CHUNK), F32)] * nch2,
    )(degp, b1.reshape(1, DH), W2, *m1, *agg1)

    agg2 = _make_agg(NP, nbt, nch2)(src, dst, zerosC, *m2)

    outp = pl.pallas_call(
        functools.partial(_mm3_body, nch2),
        grid=grid,
        in_specs=[degp_spec,
                  pl.BlockSpec((1, DO), lambda i: (0, 0))]
                 + [chunk_spec] * nch2 + [agg_spec] * nch2,
        out_specs=pl.BlockSpec((R, DO), lambda i: (i, 0)),
        out_shape=jax.ShapeDtypeStruct((NP, DO), F32),
    )(degp, b2.reshape(1, DO), *m2, *agg2)

    return outp[:N]


# speedup vs baseline: 2.6043x; 1.0633x over previous
import functools

import jax
import jax.numpy as jnp
from jax import lax
from jax.experimental import pallas as pl
from jax.experimental.pallas import tpu as pltpu
from jax.experimental.pallas import tpu_sc as plsc

F32 = jnp.float32
NTILES = 32
NSUB = 16
BATCH = 128
CHUNK = 128
R = 512


def _cdiv(a, b):
    return (a + b - 1) // b


def _dinv_from(degp_ref):
    dp = degp_ref[0] + degp_ref[1]
    return lax.rsqrt(1.0 + dp[:, 0:1])


def _mm1_body(degp_ref, x_ref, w_ref, *out_refs):
    dinv = _dinv_from(degp_ref)
    h = jnp.dot(x_ref[...], w_ref[...], preferred_element_type=F32)
    m = h * dinv
    for c, o in enumerate(out_refs):
        o[...] = m[:, c * CHUNK:(c + 1) * CHUNK]


def _mm2_body(nch_in, nch_out, degp_ref, b_ref, w_ref, *refs):
    m_refs = refs[:nch_in]
    a_refs = refs[nch_in:2 * nch_in]
    out_refs = refs[2 * nch_in:]
    dinv = _dinv_from(degp_ref)
    parts = [m_refs[c][...] + a_refs[c][0] + a_refs[c][1] for c in range(nch_in)]
    t = jnp.concatenate(parts, axis=1)
    y = jnp.maximum(t * dinv + b_ref[...], 0.0)
    m2 = jnp.dot(y, w_ref[...], preferred_element_type=F32) * dinv
    for c, o in enumerate(out_refs):
        o[...] = m2[:, c * CHUNK:(c + 1) * CHUNK]


def _mm3_body(nch, degp_ref, b_ref, *refs):
    m_refs = refs[:nch]
    a_refs = refs[nch:2 * nch]
    out_ref = refs[2 * nch]
    dinv = _dinv_from(degp_ref)
    parts = [m_refs[c][...] + a_refs[c][0] + a_refs[c][1] for c in range(nch)]
    t = jnp.concatenate(parts, axis=1)
    out_ref[...] = t * dinv + b_ref[...]


def _sc_mesh():
    return plsc.VectorSubcoreMesh(core_axis_name="c", subcore_axis_name="s")


def _make_deg(NP, nbt):
    KS = NP // NSUB // BATCH

    def body(dst_r, ones_hbm, zeros_hbm, out_ref, idx_d, ones_v,
             bufA, acc):
        cid = lax.axis_index("c")
        sid = lax.axis_index("s")
        wid = cid * NSUB + sid
        pltpu.sync_copy(dst_r.at[wid], idx_d)
        pltpu.sync_copy(ones_hbm, ones_v)
        pltpu.sync_copy(zeros_hbm, bufA)
        base = sid * (NP // NSUB)
        for k in range(KS):
            pltpu.sync_copy(bufA, acc.at[pl.ds(base + k * BATCH, BATCH)])
        plsc.subcore_barrier()

        def step(j, carry):
            pltpu.sync_copy(ones_v, acc.at[idx_d.at[j]], add=True)
            return carry

        lax.fori_loop(0, nbt, step, 0)
        plsc.subcore_barrier()
        for k in range(KS):
            pltpu.sync_copy(acc.at[pl.ds(base + k * BATCH, BATCH)], bufA)
            pltpu.sync_copy(bufA, out_ref.at[cid, pl.ds(base + k * BATCH, BATCH)])

    return pl.kernel(
        body,
        out_type=jax.ShapeDtypeStruct((2, NP, CHUNK), F32),
        mesh=_sc_mesh(),
        scratch_types=[
            pltpu.VMEM((nbt, BATCH), jnp.int32),
            pltpu.VMEM((BATCH, CHUNK), F32),
            pltpu.VMEM((BATCH, CHUNK), F32),
            pltpu.VMEM_SHARED((NP, CHUNK), F32),
        ],
    )


def _make_agg(NP, nbt, nch):
    KS = NP // NSUB // BATCH

    def body(src_r, dst_r, zeros_hbm, *rest):
        m_refs = rest[:nch]
        out_refs = rest[nch:2 * nch]
        idx_s, idx_d, bufA, bufB, acc, gsemA, gsemB = rest[2 * nch:]
        cid = lax.axis_index("c")
        sid = lax.axis_index("s")
        wid = cid * NSUB + sid
        pltpu.sync_copy(src_r.at[wid], idx_s)
        pltpu.sync_copy(dst_r.at[wid], idx_d)
        base = sid * (NP // NSUB)
        last = nbt - 1
        for ch in range(nch):
            m = m_refs[ch]
            pltpu.sync_copy(zeros_hbm, bufA)
            for k in range(KS):
                pltpu.sync_copy(bufA, acc.at[pl.ds(base + k * BATCH, BATCH)])
            plsc.subcore_barrier()
            pltpu.async_copy(m.at[idx_s.at[0]], bufA, gsemA)
            pltpu.async_copy(m.at[idx_s.at[1]], bufB, gsemB)

            def pair(gp, carry, _m=m):
                j0 = 2 * gp
                pltpu.make_async_copy(_m.at[idx_s.at[j0]], bufA, gsemA).wait()
                pltpu.sync_copy(bufA, acc.at[idx_d.at[j0]], add=True)
                pltpu.async_copy(_m.at[idx_s.at[lax.min(j0 + 2, last)]],
                                 bufA, gsemA)
                pltpu.make_async_copy(_m.at[idx_s.at[j0 + 1]], bufB, gsemB).wait()
                pltpu.sync_copy(bufB, acc.at[idx_d.at[j0 + 1]], add=True)
                pltpu.async_copy(_m.at[idx_s.at[lax.min(j0 + 3, last)]],
                                 bufB, gsemB)
                return carry

            lax.fori_loop(0, nbt // 2, pair, 0)
            pltpu.make_async_copy(m.at[idx_s.at[last]], bufA, gsemA).wait()
            pltpu.make_async_copy(m.at[idx_s.at[last]], bufB, gsemB).wait()
            plsc.subcore_barrier()
            for k in range(KS):
                pltpu.sync_copy(acc.at[pl.ds(base + k * BATCH, BATCH)], bufA)
                pltpu.sync_copy(
                    bufA, out_refs[ch].at[cid, pl.ds(base + k * BATCH, BATCH)])
            plsc.subcore_barrier()

    return pl.kernel(
        body,
        out_type=[jax.ShapeDtypeStruct((2, NP, CHUNK), F32)] * nch,
        mesh=_sc_mesh(),
        scratch_types=[
            pltpu.VMEM((nbt, BATCH), jnp.int32),
            pltpu.VMEM((nbt, BATCH), jnp.int32),
            pltpu.VMEM((BATCH, CHUNK), F32),
            pltpu.VMEM((BATCH, CHUNK), F32),
            pltpu.VMEM_SHARED((NP, CHUNK), F32),
            pltpu.SemaphoreType.DMA,
            pltpu.SemaphoreType.DMA,
        ],
    )


def kernel(x, edge_index, W1, b1, W2, b2):
    N, DI = x.shape
    DH = W1.shape[1]
    DO = W2.shape[1]
    E = edge_index.shape[1]
    nch1 = DH // CHUNK
    nch2 = DO // CHUNK

    NP = _cdiv(N + 1, NSUB * BATCH) * (NSUB * BATCH)
    nbt = 2 * _cdiv(E, NTILES * BATCH * 2)
    EP = NTILES * BATCH * nbt
    grid = (NP // R,)

    src = jnp.concatenate([edge_index[0],
                           jnp.full((EP - E,), N, jnp.int32)]).reshape(
                               NTILES, nbt, BATCH)
    dst = jnp.concatenate([edge_index[1],
                           jnp.full((EP - E,), N, jnp.int32)]).reshape(
                               NTILES, nbt, BATCH)
    xp = jnp.pad(x, ((0, NP - N), (0, 0)))
    zerosC = jnp.zeros((BATCH, CHUNK), F32)
    onesC = jnp.ones((BATCH, CHUNK), F32)

    degp = _make_deg(NP, nbt)(dst, onesC, zerosC)


    degp_spec = pl.BlockSpec((2, R, CHUNK), lambda i: (0, i, 0))
    chunk_spec = pl.BlockSpec((R, CHUNK), lambda i: (i, 0))
    agg_spec = pl.BlockSpec((2, R, CHUNK), lambda i: (0, i, 0))
    m1 = pl.pallas_call(
        _mm1_body,
        grid=grid,
        in_specs=[degp_spec,
                  pl.BlockSpec((R, DI), lambda i: (i, 0)),
                  pl.BlockSpec((DI, DH), lambda i: (0, 0))],
        out_specs=[chunk_spec] * nch1,
        out_shape=[jax.ShapeDtypeStruct((NP, CHUNK), F32)] * nch1,
    )(degp, xp, W1)

    agg1 = _make_agg(NP, nbt, nch1)(src, dst, zerosC, *m1)

    m2 = pl.pallas_call(
        functools.partial(_mm2_body, nch1, nch2),
        grid=grid,
        in_specs=[degp_spec,
                  pl.BlockSpec((1, DH), lambda i: (0, 0)),
                  pl.BlockSpec((DH, DO), lambda i: (0, 0))]
                 + [chunk_spec] * nch1 + [agg_spec] * nch1,
        out_specs=[chunk_spec] * nch2,
        out_shape=[jax.ShapeDtypeStruct((NP, CHUNK), F32)] * nch2,
    )(degp, b1.reshape(1, DH), W2, *m1, *agg1)

    agg2 = _make_agg(NP, nbt, nch2)(src, dst, zerosC, *m2)

    outp = pl.pallas_call(
        functools.partial(_mm3_body, nch2),
        grid=grid,
        in_specs=[degp_spec,
                  pl.BlockSpec((1, DO), lambda i: (0, 0))]
                 + [chunk_spec] * nch2 + [agg_spec] * nch2,
        out_specs=pl.BlockSpec((R, DO), lambda i: (i, 0)),
        out_shape=jax.ShapeDtypeStruct((NP, DO), F32),
    )(degp, b2.reshape(1, DO), *m2, *agg2)

    return outp[:N]


# speedup vs baseline: 2.6525x; 1.0185x over previous
import functools

import jax
import jax.numpy as jnp
from jax import lax
from jax.experimental import pallas as pl
from jax.experimental.pallas import tpu as pltpu
from jax.experimental.pallas import tpu_sc as plsc

F32 = jnp.float32
NTILES = 32
NSUB = 16
BATCH = 128
CHUNK = 128
R = 512


def _cdiv(a, b):
    return (a + b - 1) // b


def _dinv_from(degp_ref):
    dp = degp_ref[0] + degp_ref[1]
    return lax.rsqrt(1.0 + dp[:, 0:1])


def _mm1_body(degp_ref, x_ref, w_ref, *out_refs):
    dinv = _dinv_from(degp_ref)
    h = jnp.dot(x_ref[...], w_ref[...], preferred_element_type=F32)
    m = h * dinv
    for c, o in enumerate(out_refs):
        o[...] = m[:, c * CHUNK:(c + 1) * CHUNK]


def _mm2_body(nch_in, nch_out, degp_ref, b_ref, w_ref, *refs):
    m_refs = refs[:nch_in]
    a_refs = refs[nch_in:2 * nch_in]
    out_refs = refs[2 * nch_in:]
    dinv = _dinv_from(degp_ref)
    parts = [m_refs[c][...] + a_refs[c][0] + a_refs[c][1] for c in range(nch_in)]
    t = jnp.concatenate(parts, axis=1)
    y = jnp.maximum(t * dinv + b_ref[...], 0.0)
    m2 = jnp.dot(y, w_ref[...], preferred_element_type=F32) * dinv
    for c, o in enumerate(out_refs):
        o[...] = m2[:, c * CHUNK:(c + 1) * CHUNK]


def _mm3_body(nch, degp_ref, b_ref, *refs):
    m_refs = refs[:nch]
    a_refs = refs[nch:2 * nch]
    out_ref = refs[2 * nch]
    dinv = _dinv_from(degp_ref)
    parts = [m_refs[c][...] + a_refs[c][0] + a_refs[c][1] for c in range(nch)]
    t = jnp.concatenate(parts, axis=1)
    out_ref[...] = t * dinv + b_ref[...]


def _sc_mesh():
    return plsc.VectorSubcoreMesh(core_axis_name="c", subcore_axis_name="s")


def _make_deg(NP, nbt):
    KS = NP // NSUB // BATCH

    def body(dst_r, ones_hbm, zeros_hbm, out_ref, idx_d, ones_v,
             bufA, acc):
        cid = lax.axis_index("c")
        sid = lax.axis_index("s")
        wid = cid * NSUB + sid
        pltpu.sync_copy(dst_r.at[pl.ds(wid * nbt, nbt)], idx_d)
        pltpu.sync_copy(ones_hbm, ones_v)
        pltpu.sync_copy(zeros_hbm, bufA)
        base = sid * (NP // NSUB)
        for k in range(KS):
            pltpu.sync_copy(bufA, acc.at[pl.ds(base + k * BATCH, BATCH)])
        plsc.subcore_barrier()

        def step(j, carry):
            pltpu.sync_copy(ones_v, acc.at[idx_d.at[j]], add=True)
            return carry

        lax.fori_loop(0, nbt, step, 0)
        plsc.subcore_barrier()
        for k in range(KS):
            pltpu.sync_copy(acc.at[pl.ds(base + k * BATCH, BATCH)], bufA)
            pltpu.sync_copy(bufA, out_ref.at[cid, pl.ds(base + k * BATCH, BATCH)])

    return pl.kernel(
        body,
        out_type=jax.ShapeDtypeStruct((2, NP, CHUNK), F32),
        mesh=_sc_mesh(),
        scratch_types=[
            pltpu.VMEM((nbt, BATCH), jnp.int32),
            pltpu.VMEM((BATCH, CHUNK), F32),
            pltpu.VMEM((BATCH, CHUNK), F32),
            pltpu.VMEM_SHARED((NP, CHUNK), F32),
        ],
    )


def _make_agg(NP, nb0, nb1, nch):
    ACCROWS = 10112
    RPT = ACCROWS // NSUB
    KSF, KREM = RPT // BATCH, RPT % BATCH

    def body(src_r, dst_r, zeros_hbm, *rest):
        m_refs = rest[:nch]
        out_refs = rest[nch:2 * nch]
        idx_s, idx_d, bufA, bufB, acc, gsemA, gsemB = rest[2 * nch:]
        cid = lax.axis_index("c")
        sid = lax.axis_index("s")
        start = lax.select(cid == 0, sid * nb0, NSUB * nb0 + sid * nb1)
        cnt = lax.select(cid == 0, nb0, nb1)
        pairs = lax.select(cid == 0, nb0 // 2, nb1 // 2)
        last = cnt - 1
        pltpu.sync_copy(src_r.at[pl.ds(start, nb0)], idx_s)
        pltpu.sync_copy(dst_r.at[pl.ds(start, nb0)], idx_d)
        base = sid * RPT
        for ch in range(nch):
            m = m_refs[ch]
            pltpu.sync_copy(zeros_hbm, bufA)
            for k in range(KSF):
                pltpu.sync_copy(bufA, acc.at[pl.ds(base + k * BATCH, BATCH)])
            if KREM:
                pltpu.sync_copy(bufA.at[pl.ds(0, KREM)],
                                acc.at[pl.ds(base + KSF * BATCH, KREM)])
            plsc.subcore_barrier()
            pltpu.async_copy(m.at[idx_s.at[0]], bufA, gsemA)
            pltpu.async_copy(m.at[idx_s.at[1]], bufB, gsemB)

            def pair(gp, carry, _m=m):
                j0 = 2 * gp
                pltpu.make_async_copy(_m.at[idx_s.at[j0]], bufA, gsemA).wait()
                pltpu.sync_copy(bufA, acc.at[idx_d.at[j0]], add=True)
                pltpu.async_copy(_m.at[idx_s.at[lax.min(j0 + 2, last)]],
                                 bufA, gsemA)
                pltpu.make_async_copy(_m.at[idx_s.at[j0 + 1]], bufB, gsemB).wait()
                pltpu.sync_copy(bufB, acc.at[idx_d.at[j0 + 1]], add=True)
                pltpu.async_copy(_m.at[idx_s.at[lax.min(j0 + 3, last)]],
                                 bufB, gsemB)
                return carry

            lax.fori_loop(0, pairs, pair, 0)
            pltpu.make_async_copy(m.at[idx_s.at[last]], bufA, gsemA).wait()
            pltpu.make_async_copy(m.at[idx_s.at[last]], bufB, gsemB).wait()
            plsc.subcore_barrier()
            for k in range(KSF):
                pltpu.sync_copy(acc.at[pl.ds(base + k * BATCH, BATCH)], bufA)
                pltpu.sync_copy(
                    bufA, out_refs[ch].at[cid, pl.ds(base + k * BATCH, BATCH)])
            if KREM:
                pltpu.sync_copy(acc.at[pl.ds(base + KSF * BATCH, KREM)],
                                bufA.at[pl.ds(0, KREM)])
                pltpu.sync_copy(
                    bufA.at[pl.ds(0, KREM)],
                    out_refs[ch].at[cid, pl.ds(base + KSF * BATCH, KREM)])
            plsc.subcore_barrier()

    return pl.kernel(
        body,
        out_type=[jax.ShapeDtypeStruct((2, NP, CHUNK), F32)] * nch,
        mesh=_sc_mesh(),
        scratch_types=[
            pltpu.VMEM((nb0, BATCH), jnp.int32),
            pltpu.VMEM((nb0, BATCH), jnp.int32),
            pltpu.VMEM((BATCH, CHUNK), F32),
            pltpu.VMEM((BATCH, CHUNK), F32),
            pltpu.VMEM_SHARED((ACCROWS, CHUNK), F32),
            pltpu.SemaphoreType.DMA,
            pltpu.SemaphoreType.DMA,
        ],
    )


def kernel(x, edge_index, W1, b1, W2, b2):
    N, DI = x.shape
    DH = W1.shape[1]
    DO = W2.shape[1]
    E = edge_index.shape[1]
    nch1 = DH // CHUNK
    nch2 = DO // CHUNK

    NP = _cdiv(N + 1, NSUB * BATCH) * (NSUB * BATCH)
    nbt = 2 * _cdiv(E, NTILES * BATCH * 2)
    EP = NTILES * BATCH * nbt
    grid = (NP // R,)

    nb0 = (7 * NTILES * nbt // 10) // (8 * NSUB) * 8
    nb1 = NTILES * nbt // NSUB - nb0
    npad = nb0 - nb1
    src = jnp.concatenate([edge_index[0],
                           jnp.full((EP - E + npad * BATCH,), N,
                                    jnp.int32)]).reshape(-1, BATCH)
    dst = jnp.concatenate([edge_index[1],
                           jnp.full((EP - E + npad * BATCH,), N,
                                    jnp.int32)]).reshape(-1, BATCH)
    xp = jnp.pad(x, ((0, NP - N), (0, 0)))
    zerosC = jnp.zeros((BATCH, CHUNK), F32)
    onesC = jnp.ones((BATCH, CHUNK), F32)

    degp = _make_deg(NP, nbt)(dst, onesC, zerosC)


    degp_spec = pl.BlockSpec((2, R, CHUNK), lambda i: (0, i, 0))
    chunk_spec = pl.BlockSpec((R, CHUNK), lambda i: (i, 0))
    agg_spec = pl.BlockSpec((2, R, CHUNK), lambda i: (0, i, 0))
    m1 = pl.pallas_call(
        _mm1_body,
        grid=grid,
        in_specs=[degp_spec,
                  pl.BlockSpec((R, DI), lambda i: (i, 0)),
                  pl.BlockSpec((DI, DH), lambda i: (0, 0))],
        out_specs=[chunk_spec] * nch1,
        out_shape=[jax.ShapeDtypeStruct((NP, CHUNK), F32)] * nch1,
    )(degp, xp, W1)

    agg1 = _make_agg(NP, nb0, nb1, nch1)(src, dst, zerosC, *m1)

    m2 = pl.pallas_call(
        functools.partial(_mm2_body, nch1, nch2),
        grid=grid,
        in_specs=[degp_spec,
                  pl.BlockSpec((1, DH), lambda i: (0, 0)),
                  pl.BlockSpec((DH, DO), lambda i: (0, 0))]
                 + [chunk_spec] * nch1 + [agg_spec] * nch1,
        out_specs=[chunk_spec] * nch2,
        out_shape=[jax.ShapeDtypeStruct((NP, CHUNK), F32)] * nch2,
    )(degp, b1.reshape(1, DH), W2, *m1, *agg1)

    agg2 = _make_agg(NP, nb0, nb1, nch2)(src, dst, zerosC, *m2)

    outp = pl.pallas_call(
        functools.partial(_mm3_body, nch2),
        grid=grid,
        in_specs=[degp_spec,
                  pl.BlockSpec((1, DO), lambda i: (0, 0))]
                 + [chunk_spec] * nch2 + [agg_spec] * nch2,
        out_specs=pl.BlockSpec((R, DO), lambda i: (i, 0)),
        out_shape=jax.ShapeDtypeStruct((NP, DO), F32),
    )(degp, b2.reshape(1, DO), *m2, *agg2)

    return outp[:N]


# speedup vs baseline: 2.6677x; 1.0057x over previous
import functools

import jax
import jax.numpy as jnp
from jax import lax
from jax.experimental import pallas as pl
from jax.experimental.pallas import tpu as pltpu
from jax.experimental.pallas import tpu_sc as plsc

F32 = jnp.float32
NTILES = 32
NSUB = 16
BATCH = 128
CHUNK = 128
R = 512


def _cdiv(a, b):
    return (a + b - 1) // b


def _dinv_from(degp_ref):
    dp = degp_ref[0] + degp_ref[1]
    return lax.rsqrt(1.0 + dp[:, 0:1])


def _mm1_body(degp_ref, x_ref, w_ref, *out_refs):
    dinv = _dinv_from(degp_ref)
    h = jnp.dot(x_ref[...], w_ref[...], preferred_element_type=F32)
    m = h * dinv
    for c, o in enumerate(out_refs):
        o[...] = m[:, c * CHUNK:(c + 1) * CHUNK]


def _mm2_body(nch_in, nch_out, degp_ref, b_ref, w_ref, *refs):
    m_refs = refs[:nch_in]
    a_refs = refs[nch_in:2 * nch_in]
    out_refs = refs[2 * nch_in:]
    dinv = _dinv_from(degp_ref)
    parts = [m_refs[c][...] + a_refs[c][0] + a_refs[c][1] for c in range(nch_in)]
    t = jnp.concatenate(parts, axis=1)
    y = jnp.maximum(t * dinv + b_ref[...], 0.0)
    m2 = jnp.dot(y, w_ref[...], preferred_element_type=F32) * dinv
    for c, o in enumerate(out_refs):
        o[...] = m2[:, c * CHUNK:(c + 1) * CHUNK]


def _mm3_body(nch, degp_ref, b_ref, *refs):
    m_refs = refs[:nch]
    a_refs = refs[nch:2 * nch]
    out_ref = refs[2 * nch]
    dinv = _dinv_from(degp_ref)
    parts = [m_refs[c][...] + a_refs[c][0] + a_refs[c][1] for c in range(nch)]
    t = jnp.concatenate(parts, axis=1)
    out_ref[...] = t * dinv + b_ref[...]


def _sc_mesh():
    return plsc.VectorSubcoreMesh(core_axis_name="c", subcore_axis_name="s")


def _make_deg(NP, nbt):
    KS = NP // NSUB // BATCH

    def body(dst_r, ones_hbm, zeros_hbm, out_ref, idx_d, ones_v,
             bufA, acc):
        cid = lax.axis_index("c")
        sid = lax.axis_index("s")
        wid = cid * NSUB + sid
        pltpu.sync_copy(dst_r.at[pl.ds(wid * nbt, nbt)], idx_d)
        pltpu.sync_copy(ones_hbm, ones_v)
        pltpu.sync_copy(zeros_hbm, bufA)
        base = sid * (NP // NSUB)
        for k in range(KS):
            pltpu.sync_copy(bufA, acc.at[pl.ds(base + k * BATCH, BATCH)])
        plsc.subcore_barrier()

        def step(j, carry):
            pltpu.sync_copy(ones_v, acc.at[idx_d.at[j]], add=True)
            return carry

        lax.fori_loop(0, nbt, step, 0)
        plsc.subcore_barrier()
        for k in range(KS):
            pltpu.sync_copy(acc.at[pl.ds(base + k * BATCH, BATCH)], bufA)
            pltpu.sync_copy(bufA, out_ref.at[cid, pl.ds(base + k * BATCH, BATCH)])

    return pl.kernel(
        body,
        out_type=jax.ShapeDtypeStruct((2, NP, CHUNK), F32),
        mesh=_sc_mesh(),
        scratch_types=[
            pltpu.VMEM((nbt, BATCH), jnp.int32),
            pltpu.VMEM((BATCH, CHUNK), F32),
            pltpu.VMEM((BATCH, CHUNK), F32),
            pltpu.VMEM_SHARED((NP, CHUNK), F32),
        ],
    )


def _make_agg(NP, nb0, nb1, nch):
    ACCROWS = 10112
    RPT = ACCROWS // NSUB
    KSF, KREM = RPT // BATCH, RPT % BATCH

    def body(src_r, dst_r, zeros_hbm, *rest):
        m_refs = rest[:nch]
        out_refs = rest[nch:2 * nch]
        idx_s, idx_d, bufA, bufB, acc, gsemA, gsemB = rest[2 * nch:]
        cid = lax.axis_index("c")
        sid = lax.axis_index("s")
        start = lax.select(cid == 0, sid * nb0, NSUB * nb0 + sid * nb1)
        cnt = lax.select(cid == 0, nb0, nb1)
        pairs = lax.select(cid == 0, nb0 // 2, nb1 // 2)
        last = cnt - 1
        pltpu.sync_copy(src_r.at[pl.ds(start, nb0)], idx_s)
        pltpu.sync_copy(dst_r.at[pl.ds(start, nb0)], idx_d)
        base = sid * RPT
        for ch in range(nch):
            m = m_refs[ch]
            pltpu.sync_copy(zeros_hbm, bufA)
            for k in range(KSF):
                pltpu.sync_copy(bufA, acc.at[pl.ds(base + k * BATCH, BATCH)])
            if KREM:
                pltpu.sync_copy(bufA.at[pl.ds(0, KREM)],
                                acc.at[pl.ds(base + KSF * BATCH, KREM)])
            plsc.subcore_barrier()
            pltpu.async_copy(m.at[idx_s.at[0]], bufA, gsemA)
            pltpu.async_copy(m.at[idx_s.at[1]], bufB, gsemB)

            def pair(gp, carry, _m=m):
                j0 = 2 * gp
                pltpu.make_async_copy(_m.at[idx_s.at[j0]], bufA, gsemA).wait()
                pltpu.sync_copy(bufA, acc.at[idx_d.at[j0]], add=True)
                pltpu.async_copy(_m.at[idx_s.at[lax.min(j0 + 2, last)]],
                                 bufA, gsemA)
                pltpu.make_async_copy(_m.at[idx_s.at[j0 + 1]], bufB, gsemB).wait()
                pltpu.sync_copy(bufB, acc.at[idx_d.at[j0 + 1]], add=True)
                pltpu.async_copy(_m.at[idx_s.at[lax.min(j0 + 3, last)]],
                                 bufB, gsemB)
                return carry

            lax.fori_loop(0, pairs, pair, 0)
            pltpu.make_async_copy(m.at[idx_s.at[last]], bufA, gsemA).wait()
            pltpu.make_async_copy(m.at[idx_s.at[last]], bufB, gsemB).wait()
            plsc.subcore_barrier()
            for k in range(KSF):
                pltpu.sync_copy(acc.at[pl.ds(base + k * BATCH, BATCH)], bufA)
                pltpu.sync_copy(
                    bufA, out_refs[ch].at[cid, pl.ds(base + k * BATCH, BATCH)])
            if KREM:
                pltpu.sync_copy(acc.at[pl.ds(base + KSF * BATCH, KREM)],
                                bufA.at[pl.ds(0, KREM)])
                pltpu.sync_copy(
                    bufA.at[pl.ds(0, KREM)],
                    out_refs[ch].at[cid, pl.ds(base + KSF * BATCH, KREM)])
            plsc.subcore_barrier()

    return pl.kernel(
        body,
        out_type=[jax.ShapeDtypeStruct((2, NP, CHUNK), F32)] * nch,
        mesh=_sc_mesh(),
        scratch_types=[
            pltpu.VMEM((nb0, BATCH), jnp.int32),
            pltpu.VMEM((nb0, BATCH), jnp.int32),
            pltpu.VMEM((BATCH, CHUNK), F32),
            pltpu.VMEM((BATCH, CHUNK), F32),
            pltpu.VMEM_SHARED((ACCROWS, CHUNK), F32),
            pltpu.SemaphoreType.DMA,
            pltpu.SemaphoreType.DMA,
        ],
    )


def kernel(x, edge_index, W1, b1, W2, b2):
    N, DI = x.shape
    DH = W1.shape[1]
    DO = W2.shape[1]
    E = edge_index.shape[1]
    nch1 = DH // CHUNK
    nch2 = DO // CHUNK

    NP = _cdiv(N + 1, NSUB * BATCH) * (NSUB * BATCH)
    nbt = 2 * _cdiv(E, NTILES * BATCH * 2)
    EP = NTILES * BATCH * nbt
    grid = (NP // R,)

    nb0 = (4 * NTILES * nbt // 5) // (8 * NSUB) * 8
    nb1 = NTILES * nbt // NSUB - nb0
    npad = nb0 - nb1
    src = jnp.concatenate([edge_index[0],
                           jnp.full((EP - E + npad * BATCH,), N,
                                    jnp.int32)]).reshape(-1, BATCH)
    dst = jnp.concatenate([edge_index[1],
                           jnp.full((EP - E + npad * BATCH,), N,
                                    jnp.int32)]).reshape(-1, BATCH)
    xp = jnp.pad(x, ((0, NP - N), (0, 0)))
    zerosC = jnp.zeros((BATCH, CHUNK), F32)
    onesC = jnp.ones((BATCH, CHUNK), F32)

    degp = _make_deg(NP, nbt)(dst, onesC, zerosC)


    degp_spec = pl.BlockSpec((2, R, CHUNK), lambda i: (0, i, 0))
    chunk_spec = pl.BlockSpec((R, CHUNK), lambda i: (i, 0))
    agg_spec = pl.BlockSpec((2, R, CHUNK), lambda i: (0, i, 0))
    m1 = pl.pallas_call(
        _mm1_body,
        grid=grid,
        in_specs=[degp_spec,
                  pl.BlockSpec((R, DI), lambda i: (i, 0)),
                  pl.BlockSpec((DI, DH), lambda i: (0, 0))],
        out_specs=[chunk_spec] * nch1,
        out_shape=[jax.ShapeDtypeStruct((NP, CHUNK), F32)] * nch1,
    )(degp, xp, W1)

    agg1 = _make_agg(NP, nb0, nb1, nch1)(src, dst, zerosC, *m1)

    m2 = pl.pallas_call(
        functools.partial(_mm2_body, nch1, nch2),
        grid=grid,
        in_specs=[degp_spec,
                  pl.BlockSpec((1, DH), lambda i: (0, 0)),
                  pl.BlockSpec((DH, DO), lambda i: (0, 0))]
                 + [chunk_spec] * nch1 + [agg_spec] * nch1,
        out_specs=[chunk_spec] * nch2,
        out_shape=[jax.ShapeDtypeStruct((NP, CHUNK), F32)] * nch2,
    )(degp, b1.reshape(1, DH), W2, *m1, *agg1)

    agg2 = _make_agg(NP, nb0, nb1, nch2)(src, dst, zerosC, *m2)

    outp = pl.pallas_call(
        functools.partial(_mm3_body, nch2),
        grid=grid,
        in_specs=[degp_spec,
                  pl.BlockSpec((1, DO), lambda i: (0, 0))]
                 + [chunk_spec] * nch2 + [agg_spec] * nch2,
        out_specs=pl.BlockSpec((R, DO), lambda i: (i, 0)),
        out_shape=jax.ShapeDtypeStruct((NP, DO), F32),
    )(degp, b2.reshape(1, DO), *m2, *agg2)

    return outp[:N]
